# baseline (speedup 1.0000x reference)
import functools

import jax
import jax.numpy as jnp
from jax import lax
from jax.experimental import pallas as pl
from jax.experimental.pallas import tpu as pltpu
from jax.experimental.pallas import tpu_sc as plsc

N = 10000
E = 320000
D = 128
G = 64
NC, NS, LANES = 2, 16, 16
NTILES = NC * NS
CHUNK = 128
CPT = 80
EP = NTILES * CPT * CHUNK
ROWS = 10240
RPT = ROWS // NS
PAD_ROW = N

_mesh = plsc.VectorSubcoreMesh(core_axis_name="c", subcore_axis_name="s")
_sc_params = pltpu.CompilerParams(use_tc_tiling_on_sc=False)


@functools.partial(
    pl.kernel,
    out_type=jax.ShapeDtypeStruct((NC, ROWS, LANES), jnp.float32),
    mesh=_mesh,
    scratch_types=[
        pltpu.VMEM((CPT, CHUNK), jnp.int32),
        pltpu.VMEM((CHUNK, LANES), jnp.float32),
        pltpu.VMEM_SHARED((ROWS, LANES), jnp.float32),
        pltpu.SemaphoreType.DMA,
    ],
    compiler_params=_sc_params,
)
def _sc_deg(dst_hbm, ones_hbm, zeros_hbm, out_hbm, dstv, onesv, acc, sem):
    c = lax.axis_index("c")
    s = lax.axis_index("s")
    t = s * NC + c
    pltpu.async_copy(dst_hbm.at[t], dstv, sem).wait()
    pltpu.async_copy(ones_hbm, onesv, sem).wait()
    pltpu.async_copy(zeros_hbm, acc.at[pl.ds(s * RPT, RPT)], sem).wait()
    plsc.subcore_barrier()

    @pl.loop(0, CPT)
    def _(j):
        pltpu.sync_copy(onesv, acc.at[dstv.at[j]], add=True)

    plsc.subcore_barrier()
    pltpu.sync_copy(acc.at[pl.ds(s * RPT, RPT)],
                    out_hbm.at[c, pl.ds(s * RPT, RPT)])


@functools.partial(
    pl.kernel,
    out_type=jax.ShapeDtypeStruct((NC, ROWS, D), jnp.bfloat16),
    mesh=_mesh,
    scratch_types=[
        pltpu.VMEM((CPT, CHUNK), jnp.int32),
        pltpu.VMEM((CPT, CHUNK), jnp.int32),
        pltpu.VMEM((CHUNK, D), jnp.bfloat16),
        pltpu.VMEM((CHUNK, D), jnp.bfloat16),
        pltpu.VMEM((CHUNK, D), jnp.bfloat16),
        pltpu.VMEM((CHUNK, D), jnp.bfloat16),
        pltpu.VMEM_SHARED((ROWS, D), jnp.bfloat16),
        pltpu.SemaphoreType.DMA,
        pltpu.SemaphoreType.DMA,
        pltpu.SemaphoreType.DMA,
        pltpu.SemaphoreType.DMA,
        pltpu.SemaphoreType.DMA,
        pltpu.SemaphoreType.DMA,
        pltpu.SemaphoreType.DMA,
        pltpu.SemaphoreType.DMA,
        pltpu.SemaphoreType.DMA,
    ],
    compiler_params=_sc_params,
)
def _sc_agg(g_hbm, src_hbm, dst_hbm, zeros_hbm, out_hbm,
            srcv, dstv, rows0, rows1, rows2, rows3, acc,
            sg0, sg1, sg2, sg3, ss0, ss1, ss2, ss3, semz):
    c = lax.axis_index("c")
    s = lax.axis_index("s")
    bufs = (rows0, rows1, rows2, rows3)
    sgs = (sg0, sg1, sg2, sg3)
    sss = (ss0, ss1, ss2, ss3)
    pltpu.async_copy(zeros_hbm, acc.at[pl.ds(s * RPT, RPT)], semz).wait()
    plsc.subcore_barrier()

    @pl.when(c == 1)
    def _():
        for pas in range(2):
            t = s * NC + pas
            pltpu.async_copy(src_hbm.at[t], srcv, semz).wait()
            pltpu.async_copy(dst_hbm.at[t], dstv, semz).wait()

            @pl.loop(0, CPT + 4, step=4)
            def _(j):
                for k in range(4):
                    ci = j + k
                    cc = ci - 2
                    kc = (k + 2) % 4

                    @pl.when(ci >= 4)
                    def _():
                        pltpu.make_async_copy(
                            bufs[k], acc.at[dstv.at[ci - 4]], sss[k]).wait()

                    @pl.when(ci < CPT)
                    def _():
                        pltpu.async_copy(
                            g_hbm.at[srcv.at[ci]], bufs[k], sgs[k])

                    @pl.when((cc >= 0) & (cc < CPT))
                    def _():
                        pltpu.make_async_copy(
                            g_hbm.at[srcv.at[cc]], bufs[kc], sgs[kc]).wait()
                        pltpu.async_copy(
                            bufs[kc], acc.at[dstv.at[cc]], sss[kc], add=True)

    plsc.subcore_barrier()
    pltpu.sync_copy(acc.at[pl.ds(s * RPT, RPT)],
                    out_hbm.at[c, pl.ds(s * RPT, RPT)])


def _mm_k(x_ref, w_ref, o_ref):
    o_ref[...] = jnp.dot(x_ref[...], w_ref[...],
                         preferred_element_type=jnp.float32)


def _mm(x, w):
    return pl.pallas_call(
        _mm_k,
        out_shape=jax.ShapeDtypeStruct((x.shape[0], w.shape[1]), jnp.float32),
    )(x, w)


def _prep_k(degp_ref, hw_ref, dinv_ref, g_ref):
    d = degp_ref[0, :N, 0:1] + degp_ref[1, :N, 0:1] + 1.0
    dinv = jax.lax.rsqrt(d)
    dinvb = jnp.broadcast_to(dinv, (N, D))
    dinv_ref[...] = dinvb
    g_ref[...] = (dinvb * hw_ref[...]).astype(jnp.bfloat16)


def _prep(degp, hw):
    return pl.pallas_call(
        _prep_k,
        out_shape=[jax.ShapeDtypeStruct((N, D), jnp.float32),
                   jax.ShapeDtypeStruct((N, D), jnp.bfloat16)],
    )(degp, hw)


def _agg_full(s_ref, g_ref):
    s0 = s_ref[0, :N, :].astype(jnp.float32)
    s1 = s_ref[1, :N, :].astype(jnp.float32)
    return s0 + s1 + g_ref[...].astype(jnp.float32)


def _layer_k(s_ref, g_ref, dinv_ref, b_ref, w_ref, gout_ref):
    agg = _agg_full(s_ref, g_ref)
    dinv = dinv_ref[...]
    h = jnp.maximum(dinv * agg + b_ref[...], 0.0)
    gout_ref[...] = (dinv * jnp.dot(h, w_ref[...],
                                    preferred_element_type=jnp.float32)
                     ).astype(jnp.bfloat16)


def _layer(s, g, dinvb, b, w):
    return pl.pallas_call(
        _layer_k,
        out_shape=jax.ShapeDtypeStruct((N, D), jnp.bfloat16),
    )(s, g, dinvb, b, w)


def _head_k(s_ref, g_ref, dinv_ref, b_ref, batch_ref,
            wc1_ref, bc1_ref, wc2_ref, bc2_ref, out_ref):
    agg = _agg_full(s_ref, g_ref)
    h = jnp.maximum(dinv_ref[...] * agg + b_ref[...], 0.0)
    b = batch_ref[...]
    gids = jax.lax.broadcasted_iota(jnp.int32, (G, N), 0)
    oh = (b[None, :] == gids).astype(jnp.float32)
    sums = jnp.dot(oh, h, preferred_element_type=jnp.float32)
    counts = jnp.sum(oh, axis=1, keepdims=True)
    pooled = sums / jnp.maximum(counts, 1.0)
    z = jnp.maximum(
        jnp.dot(pooled, wc1_ref[...], preferred_element_type=jnp.float32)
        + bc1_ref[...], 0.0)
    logits = (jnp.dot(z, wc2_ref[...], preferred_element_type=jnp.float32)
              + bc2_ref[...])
    m = jnp.max(logits, axis=1, keepdims=True)
    lse = jnp.log(jnp.sum(jnp.exp(logits - m), axis=1, keepdims=True)) + m
    out_ref[...] = logits - lse


def _head(s, g, dinvb, b, batch, wc1, bc1, wc2, bc2):
    return pl.pallas_call(
        _head_k,
        out_shape=jax.ShapeDtypeStruct((G, 2), jnp.float32),
    )(s, g, dinvb, b, batch, wc1, bc1, wc2, bc2)


def kernel(x, edge_index, batch, W1, b1, W2, b2, W3, b3, Wc1, bc1, Wc2, bc2):
    src = edge_index[0].astype(jnp.int32)
    dst = edge_index[1].astype(jnp.int32)
    src_p = jnp.concatenate(
        [src, jnp.zeros((EP - E,), jnp.int32)]
    ).reshape(NTILES, CPT, CHUNK)
    pad_dst = PAD_ROW + jnp.arange(EP - E, dtype=jnp.int32) % (ROWS - N)
    dst_p = jnp.concatenate([dst, pad_dst]).reshape(NTILES, CPT, CHUNK)
    ones16 = jnp.ones((CHUNK, LANES), jnp.float32)
    z16 = jnp.zeros((RPT, LANES), jnp.float32)
    zfull = jnp.zeros((RPT, D), jnp.bfloat16)

    degp = _sc_deg(dst_p, ones16, z16)
    hw1 = _mm(x, W1)
    dinvb, g1 = _prep(degp, hw1)
    s1 = _sc_agg(g1, src_p, dst_p, zfull)
    g2 = _layer(s1, g1, dinvb, b1, W2)
    s2 = _sc_agg(g2, src_p, dst_p, zfull)
    g3 = _layer(s2, g2, dinvb, b2, W3)
    s3 = _sc_agg(g3, src_p, dst_p, zfull)
    return _head(s3, g3, dinvb, b3, batch.astype(jnp.int32),
                 Wc1, bc1, Wc2, bc2)


# speedup vs baseline: 1.2277x; 1.2277x over previous
import functools

import jax
import jax.numpy as jnp
from jax import lax
from jax.experimental import pallas as pl
from jax.experimental.pallas import tpu as pltpu
from jax.experimental.pallas import tpu_sc as plsc

N = 10000
E = 320000
D = 128
HD = 64
G = 64
NC, NS, LANES = 2, 16, 16
CHUNK = 128
CHUNKS_PER_TILE = 160
EP = NS * CHUNKS_PER_TILE * CHUNK
ROWS = 10240
RPT = ROWS // NS
PAD_ROW = N

_mesh = plsc.VectorSubcoreMesh(core_axis_name="c", subcore_axis_name="s")
_sc_params = pltpu.CompilerParams(use_tc_tiling_on_sc=False)


@functools.partial(
    pl.kernel,
    out_type=jax.ShapeDtypeStruct((NC, ROWS, LANES), jnp.float32),
    mesh=_mesh,
    scratch_types=[
        pltpu.VMEM((CHUNKS_PER_TILE, CHUNK), jnp.int32),
        pltpu.VMEM((CHUNK, LANES), jnp.float32),
        pltpu.VMEM((RPT, LANES), jnp.float32),
        pltpu.VMEM_SHARED((ROWS, LANES), jnp.float32),
        pltpu.SemaphoreType.DMA,
    ],
    compiler_params=_sc_params,
)
def _sc_deg(dst_hbm, ones_hbm, zeros_hbm, out_hbm, dstv, onesv, ztile, acc, sem):
    c = lax.axis_index("c")
    s = lax.axis_index("s")
    pltpu.async_copy(dst_hbm.at[s], dstv, sem).wait()
    pltpu.async_copy(ones_hbm, onesv, sem).wait()
    pltpu.async_copy(zeros_hbm, ztile, sem).wait()
    pltpu.sync_copy(ztile, acc.at[pl.ds(s * RPT, RPT)])
    plsc.subcore_barrier()

    @pl.loop(0, CHUNKS_PER_TILE // 2)
    def _(j):
        pltpu.sync_copy(onesv, acc.at[dstv.at[2 * j + c]], add=True)

    plsc.subcore_barrier()
    pltpu.sync_copy(acc.at[pl.ds(s * RPT, RPT)],
                    out_hbm.at[c, pl.ds(s * RPT, RPT)])


@functools.partial(
    pl.kernel,
    out_type=jax.ShapeDtypeStruct((NC, ROWS, HD), jnp.float32),
    mesh=_mesh,
    scratch_types=[
        pltpu.VMEM((CHUNKS_PER_TILE, CHUNK), jnp.int32),
        pltpu.VMEM((CHUNKS_PER_TILE, CHUNK), jnp.int32),
        pltpu.VMEM((CHUNK, HD), jnp.float32),
        pltpu.VMEM((CHUNK, HD), jnp.float32),
        pltpu.VMEM((CHUNK, HD), jnp.float32),
        pltpu.VMEM((CHUNK, HD), jnp.float32),
        pltpu.VMEM((CHUNK, HD), jnp.float32),
        pltpu.VMEM_SHARED((ROWS, HD), jnp.float32),
        pltpu.SemaphoreType.DMA,
        pltpu.SemaphoreType.DMA,
        pltpu.SemaphoreType.DMA,
        pltpu.SemaphoreType.DMA,
        pltpu.SemaphoreType.DMA,
        pltpu.SemaphoreType.DMA,
        pltpu.SemaphoreType.DMA,
        pltpu.SemaphoreType.DMA,
        pltpu.SemaphoreType.DMA,
        pltpu.SemaphoreType.DMA,
        pltpu.SemaphoreType.DMA,
    ],
    compiler_params=_sc_params,
)
def _sc_agg(gh_hbm, src_hbm, dst_hbm, zrow_hbm, out_hbm,
            srcv, dstv, rows0, rows1, rows2, rows3, rows4, acc,
            sg0, sg1, sg2, sg3, sg4, ss0, ss1, ss2, ss3, ss4, semz):
    c = lax.axis_index("c")
    s = lax.axis_index("s")
    table = gh_hbm.at[c]
    bufs = (rows0, rows1, rows2, rows3, rows4)
    sgs = (sg0, sg1, sg2, sg3, sg4)
    sss = (ss0, ss1, ss2, ss3, ss4)
    pltpu.async_copy(src_hbm.at[s], srcv, semz).wait()
    pltpu.async_copy(dst_hbm.at[s], dstv, semz).wait()
    pltpu.async_copy(zrow_hbm, rows0, semz).wait()

    @pl.loop(0, RPT // CHUNK)
    def _(k):
        pltpu.sync_copy(rows0, acc.at[pl.ds(s * RPT + k * CHUNK, CHUNK)])

    plsc.subcore_barrier()

    @pl.loop(0, CHUNKS_PER_TILE + 5, step=5)
    def _(j):
        for k in range(5):
            ci = j + k
            cc = ci - 2
            kc = (k + 3) % 5

            @pl.when(ci >= 5)
            def _():
                pltpu.make_async_copy(
                    bufs[k], acc.at[dstv.at[ci - 5]], sss[k]).wait()

            @pl.when(ci < CHUNKS_PER_TILE)
            def _():
                pltpu.async_copy(table.at[srcv.at[ci]], bufs[k], sgs[k])

            @pl.when((cc >= 0) & (cc < CHUNKS_PER_TILE))
            def _():
                pltpu.make_async_copy(
                    table.at[srcv.at[cc]], bufs[kc], sgs[kc]).wait()
                pltpu.async_copy(
                    bufs[kc], acc.at[dstv.at[cc]], sss[kc], add=True)

    plsc.subcore_barrier()

    @pl.loop(0, RPT // CHUNK)
    def _(k):
        pltpu.sync_copy(acc.at[pl.ds(s * RPT + k * CHUNK, CHUNK)],
                        out_hbm.at[c, pl.ds(s * RPT + k * CHUNK, CHUNK)])


def _mm_k(x_ref, w_ref, o_ref):
    o_ref[...] = jnp.dot(x_ref[...], w_ref[...],
                         preferred_element_type=jnp.float32)


def _mm(x, w):
    return pl.pallas_call(
        _mm_k,
        out_shape=jax.ShapeDtypeStruct((x.shape[0], w.shape[1]), jnp.float32),
    )(x, w)


def _split_write(gout_ref, g):
    gout_ref[0] = g[:, :HD]
    gout_ref[1] = g[:, HD:]


def _prep_k(degp_ref, x_ref, w_ref, dinv_ref, g_ref):
    d = degp_ref[0, :N, 0:1] + degp_ref[1, :N, 0:1] + 1.0
    dinv = jax.lax.rsqrt(d)
    dinvb = jnp.broadcast_to(dinv, (N, D))
    dinv_ref[...] = dinvb
    hw = jnp.dot(x_ref[...], w_ref[...], preferred_element_type=jnp.float32)
    _split_write(g_ref, dinvb * hw)


def _prep(degp, x, w):
    return pl.pallas_call(
        _prep_k,
        out_shape=[jax.ShapeDtypeStruct((N, D), jnp.float32),
                   jax.ShapeDtypeStruct((NC, N, HD), jnp.float32)],
    )(degp, x, w)


def _merge(s_ref, g_ref):
    s_full = jnp.concatenate([s_ref[0, :N, :], s_ref[1, :N, :]], axis=1)
    g_full = jnp.concatenate([g_ref[0], g_ref[1]], axis=1)
    return s_full + g_full


def _layer_k(s_ref, g_ref, dinv_ref, b_ref, w_ref, gout_ref):
    agg = _merge(s_ref, g_ref)
    dinv = dinv_ref[...]
    h = jnp.maximum(dinv * agg + b_ref[...], 0.0)
    _split_write(gout_ref, dinv * jnp.dot(h, w_ref[...],
                                          preferred_element_type=jnp.float32))


def _layer(s, g, dinvb, b, w):
    return pl.pallas_call(
        _layer_k,
        out_shape=jax.ShapeDtypeStruct((NC, N, HD), jnp.float32),
    )(s, g, dinvb, b, w)


def _head_k(s_ref, g_ref, dinv_ref, b_ref, batch_ref,
            wc1_ref, bc1_ref, wc2_ref, bc2_ref, out_ref):
    agg = _merge(s_ref, g_ref)
    h = jnp.maximum(dinv_ref[...] * agg + b_ref[...], 0.0)
    b = batch_ref[...]
    gids = jax.lax.broadcasted_iota(jnp.int32, (G, N), 0)
    oh = (b[None, :] == gids).astype(jnp.float32)
    sums = jnp.dot(oh, h, preferred_element_type=jnp.float32)
    counts = jnp.sum(oh, axis=1, keepdims=True)
    pooled = sums / jnp.maximum(counts, 1.0)
    z = jnp.maximum(
        jnp.dot(pooled, wc1_ref[...], preferred_element_type=jnp.float32)
        + bc1_ref[...], 0.0)
    logits = (jnp.dot(z, wc2_ref[...], preferred_element_type=jnp.float32)
              + bc2_ref[...])
    m = jnp.max(logits, axis=1, keepdims=True)
    lse = jnp.log(jnp.sum(jnp.exp(logits - m), axis=1, keepdims=True)) + m
    out_ref[...] = logits - lse


def _head(s, g, dinvb, b, batch, wc1, bc1, wc2, bc2):
    return pl.pallas_call(
        _head_k,
        out_shape=jax.ShapeDtypeStruct((G, 2), jnp.float32),
    )(s, g, dinvb, b, batch, wc1, bc1, wc2, bc2)


def kernel(x, edge_index, batch, W1, b1, W2, b2, W3, b3, Wc1, bc1, Wc2, bc2):
    src = edge_index[0].astype(jnp.int32)
    dst = edge_index[1].astype(jnp.int32)
    src_p = jnp.concatenate(
        [src, jnp.zeros((EP - E,), jnp.int32)]
    ).reshape(NS, CHUNKS_PER_TILE, CHUNK)
    pad_dst = PAD_ROW + jnp.arange(EP - E, dtype=jnp.int32) % (ROWS - N)
    dst_p = jnp.concatenate([dst, pad_dst]).reshape(NS, CHUNKS_PER_TILE, CHUNK)
    ones16 = jnp.ones((CHUNK, LANES), jnp.float32)
    z16 = jnp.zeros((RPT, LANES), jnp.float32)
    zrow = jnp.zeros((CHUNK, HD), jnp.float32)

    degp = _sc_deg(dst_p, ones16, z16)
    dinvb, g1 = _prep(degp, x, W1)
    s1 = _sc_agg(g1, src_p, dst_p, zrow)
    g2 = _layer(s1, g1, dinvb, b1, W2)
    s2 = _sc_agg(g2, src_p, dst_p, zrow)
    g3 = _layer(s2, g2, dinvb, b2, W3)
    s3 = _sc_agg(g3, src_p, dst_p, zrow)
    return _head(s3, g3, dinvb, b3, batch.astype(jnp.int32),
                 Wc1, bc1, Wc2, bc2)


# speedup vs baseline: 1.2808x; 1.0433x over previous
import functools

import jax
import jax.numpy as jnp
from jax import lax
from jax.experimental import pallas as pl
from jax.experimental.pallas import tpu as pltpu
from jax.experimental.pallas import tpu_sc as plsc

N = 10000
E = 320000
D = 128
HD = 64
G = 64
NC, NS, LANES = 2, 16, 16
CHUNK = 128
CHUNKS_PER_TILE = 160
EP = NS * CHUNKS_PER_TILE * CHUNK
ROWS = 10240
RPT = ROWS // NS
PAD_ROW = N

_mesh = plsc.VectorSubcoreMesh(core_axis_name="c", subcore_axis_name="s")
_sc_params = pltpu.CompilerParams(use_tc_tiling_on_sc=False)


@functools.partial(
    pl.kernel,
    out_type=jax.ShapeDtypeStruct((NC, ROWS, LANES), jnp.float32),
    mesh=_mesh,
    scratch_types=[
        pltpu.VMEM((CHUNKS_PER_TILE, CHUNK), jnp.int32),
        pltpu.VMEM((CHUNK, LANES), jnp.float32),
        pltpu.VMEM((RPT, LANES), jnp.float32),
        pltpu.VMEM_SHARED((ROWS, LANES), jnp.float32),
        pltpu.SemaphoreType.DMA,
    ],
    compiler_params=_sc_params,
)
def _sc_deg(dst_hbm, ones_hbm, zeros_hbm, out_hbm, dstv, onesv, ztile, acc, sem):
    c = lax.axis_index("c")
    s = lax.axis_index("s")
    pltpu.async_copy(dst_hbm.at[s], dstv, sem).wait()
    pltpu.async_copy(ones_hbm, onesv, sem).wait()
    pltpu.async_copy(zeros_hbm, ztile, sem).wait()
    pltpu.sync_copy(ztile, acc.at[pl.ds(s * RPT, RPT)])
    plsc.subcore_barrier()

    @pl.loop(0, CHUNKS_PER_TILE // 2)
    def _(j):
        pltpu.sync_copy(onesv, acc.at[dstv.at[2 * j + c]], add=True)

    plsc.subcore_barrier()
    pltpu.sync_copy(acc.at[pl.ds(s * RPT, RPT)],
                    out_hbm.at[c, pl.ds(s * RPT, RPT)])


@functools.partial(
    pl.kernel,
    out_type=jax.ShapeDtypeStruct((NC, ROWS, HD), jnp.float32),
    mesh=_mesh,
    scratch_types=[
        pltpu.VMEM((CHUNKS_PER_TILE, CHUNK), jnp.int32),
        pltpu.VMEM((CHUNKS_PER_TILE, CHUNK), jnp.int32),
        pltpu.VMEM((CHUNK, HD), jnp.float32),
        pltpu.VMEM((CHUNK, HD), jnp.float32),
        pltpu.VMEM((CHUNK, HD), jnp.float32),
        pltpu.VMEM((CHUNK, HD), jnp.float32),
        pltpu.VMEM((CHUNK, HD), jnp.float32),
        pltpu.VMEM_SHARED((ROWS, HD), jnp.float32),
        pltpu.SemaphoreType.DMA,
        pltpu.SemaphoreType.DMA,
        pltpu.SemaphoreType.DMA,
        pltpu.SemaphoreType.DMA,
        pltpu.SemaphoreType.DMA,
        pltpu.SemaphoreType.DMA,
        pltpu.SemaphoreType.DMA,
        pltpu.SemaphoreType.DMA,
        pltpu.SemaphoreType.DMA,
        pltpu.SemaphoreType.DMA,
        pltpu.SemaphoreType.DMA,
    ],
    compiler_params=_sc_params,
)
def _sc_agg(gh_hbm, src_hbm, dst_hbm, zrow_hbm, out_hbm,
            srcv, dstv, rows0, rows1, rows2, rows3, rows4, acc,
            sg0, sg1, sg2, sg3, sg4, ss0, ss1, ss2, ss3, ss4, semz):
    c = lax.axis_index("c")
    s = lax.axis_index("s")
    table = gh_hbm.at[c]
    bufs = (rows0, rows1, rows2, rows3, rows4)
    sgs = (sg0, sg1, sg2, sg3, sg4)
    sss = (ss0, ss1, ss2, ss3, ss4)
    pltpu.async_copy(src_hbm.at[s], srcv, semz).wait()
    pltpu.async_copy(dst_hbm.at[s], dstv, semz).wait()
    pltpu.async_copy(zrow_hbm, rows0, semz).wait()

    @pl.loop(0, RPT // CHUNK)
    def _(k):
        pltpu.sync_copy(rows0, acc.at[pl.ds(s * RPT + k * CHUNK, CHUNK)])

    plsc.subcore_barrier()

    @pl.loop(0, CHUNKS_PER_TILE + 5, step=5)
    def _(j):
        for k in range(5):
            ci = j + k
            cc = ci - 2
            kc = (k + 3) % 5

            @pl.when(ci >= 5)
            def _():
                pltpu.make_async_copy(
                    bufs[k], acc.at[dstv.at[ci - 5]], sss[k]).wait()

            @pl.when(ci < CHUNKS_PER_TILE)
            def _():
                pltpu.async_copy(table.at[srcv.at[ci]], bufs[k], sgs[k])

            @pl.when((cc >= 0) & (cc < CHUNKS_PER_TILE))
            def _():
                pltpu.make_async_copy(
                    table.at[srcv.at[cc]], bufs[kc], sgs[kc]).wait()
                pltpu.async_copy(
                    bufs[kc], acc.at[dstv.at[cc]], sss[kc], add=True)

    plsc.subcore_barrier()

    @pl.loop(0, RPT // CHUNK)
    def _(k):
        pltpu.sync_copy(acc.at[pl.ds(s * RPT + k * CHUNK, CHUNK)],
                        out_hbm.at[c, pl.ds(s * RPT + k * CHUNK, CHUNK)])


def _mm_k(x_ref, w_ref, o_ref):
    o_ref[...] = jnp.dot(x_ref[...], w_ref[...],
                         preferred_element_type=jnp.float32)


def _mm(x, w):
    return pl.pallas_call(
        _mm_k,
        out_shape=jax.ShapeDtypeStruct((x.shape[0], w.shape[1]), jnp.float32),
    )(x, w)


def _split_write(gout_ref, g):
    gout_ref[0] = g[:, :HD]
    gout_ref[1] = g[:, HD:]


def _prep_k(degp_ref, hw_ref, dinv_ref, g_ref):
    d = degp_ref[0, :N, 0:1] + degp_ref[1, :N, 0:1] + 1.0
    dinv = jax.lax.rsqrt(d)
    dinvb = jnp.broadcast_to(dinv, (N, D))
    dinv_ref[...] = dinvb
    _split_write(g_ref, dinvb * hw_ref[...])


def _prep(degp, hw):
    return pl.pallas_call(
        _prep_k,
        out_shape=[jax.ShapeDtypeStruct((N, D), jnp.float32),
                   jax.ShapeDtypeStruct((NC, N, HD), jnp.float32)],
    )(degp, hw)


def _merge(s_ref, g_ref):
    s_full = jnp.concatenate([s_ref[0, :N, :], s_ref[1, :N, :]], axis=1)
    g_full = jnp.concatenate([g_ref[0], g_ref[1]], axis=1)
    return s_full + g_full


def _layer_k(s_ref, g_ref, dinv_ref, b_ref, w_ref, gout_ref):
    agg = _merge(s_ref, g_ref)
    dinv = dinv_ref[...]
    h = jnp.maximum(dinv * agg + b_ref[...], 0.0)
    _split_write(gout_ref, dinv * jnp.dot(h, w_ref[...],
                                          preferred_element_type=jnp.float32))


def _layer(s, g, dinvb, b, w):
    return pl.pallas_call(
        _layer_k,
        out_shape=jax.ShapeDtypeStruct((NC, N, HD), jnp.float32),
    )(s, g, dinvb, b, w)


def _head_k(s_ref, g_ref, dinv_ref, b_ref, batch_ref,
            wc1_ref, bc1_ref, wc2_ref, bc2_ref, out_ref):
    agg = _merge(s_ref, g_ref)
    h = jnp.maximum(dinv_ref[...] * agg + b_ref[...], 0.0)
    b = batch_ref[...]
    gids = jax.lax.broadcasted_iota(jnp.int32, (G, N), 0)
    oh = (b[None, :] == gids).astype(jnp.float32)
    sums = jnp.dot(oh, h, preferred_element_type=jnp.float32)
    counts = jnp.sum(oh, axis=1, keepdims=True)
    pooled = sums / jnp.maximum(counts, 1.0)
    z = jnp.maximum(
        jnp.dot(pooled, wc1_ref[...], preferred_element_type=jnp.float32)
        + bc1_ref[...], 0.0)
    logits = (jnp.dot(z, wc2_ref[...], preferred_element_type=jnp.float32)
              + bc2_ref[...])
    m = jnp.max(logits, axis=1, keepdims=True)
    lse = jnp.log(jnp.sum(jnp.exp(logits - m), axis=1, keepdims=True)) + m
    out_ref[...] = logits - lse


def _head(s, g, dinvb, b, batch, wc1, bc1, wc2, bc2):
    return pl.pallas_call(
        _head_k,
        out_shape=jax.ShapeDtypeStruct((G, 2), jnp.float32),
    )(s, g, dinvb, b, batch, wc1, bc1, wc2, bc2)


def kernel(x, edge_index, batch, W1, b1, W2, b2, W3, b3, Wc1, bc1, Wc2, bc2):
    src = edge_index[0].astype(jnp.int32)
    dst = edge_index[1].astype(jnp.int32)
    src_p = jnp.concatenate(
        [src, jnp.zeros((EP - E,), jnp.int32)]
    ).reshape(NS, CHUNKS_PER_TILE, CHUNK)
    pad_dst = PAD_ROW + jnp.arange(EP - E, dtype=jnp.int32) % (ROWS - N)
    dst_p = jnp.concatenate([dst, pad_dst]).reshape(NS, CHUNKS_PER_TILE, CHUNK)
    ones16 = jnp.ones((CHUNK, LANES), jnp.float32)
    z16 = jnp.zeros((RPT, LANES), jnp.float32)
    zrow = jnp.zeros((CHUNK, HD), jnp.float32)

    degp = _sc_deg(dst_p, ones16, z16)
    hw1 = _mm(x, W1)
    dinvb, g1 = _prep(degp, hw1)
    s1 = _sc_agg(g1, src_p, dst_p, zrow)
    g2 = _layer(s1, g1, dinvb, b1, W2)
    s2 = _sc_agg(g2, src_p, dst_p, zrow)
    g3 = _layer(s2, g2, dinvb, b2, W3)
    s3 = _sc_agg(g3, src_p, dst_p, zrow)
    return _head(s3, g3, dinvb, b3, batch.astype(jnp.int32),
                 Wc1, bc1, Wc2, bc2)


# speedup vs baseline: 1.2923x; 1.0090x over previous
import functools

import jax
import jax.numpy as jnp
from jax import lax
from jax.experimental import pallas as pl
from jax.experimental.pallas import tpu as pltpu
from jax.experimental.pallas import tpu_sc as plsc

N = 10000
E = 320000
D = 128
HD = 64
G = 64
NC, NS, LANES = 2, 16, 16
CHUNK = 128
CHUNKS_PER_TILE = 160
EP = NS * CHUNKS_PER_TILE * CHUNK
ROWS = 10240
RPT = ROWS // NS
PAD_ROW = N

_mesh = plsc.VectorSubcoreMesh(core_axis_name="c", subcore_axis_name="s")
_sc_params = pltpu.CompilerParams(use_tc_tiling_on_sc=False)


@functools.partial(
    pl.kernel,
    out_type=jax.ShapeDtypeStruct((NC, ROWS, LANES), jnp.float32),
    mesh=_mesh,
    scratch_types=[
        pltpu.VMEM((CHUNKS_PER_TILE, CHUNK), jnp.int32),
        pltpu.VMEM((CHUNK, LANES), jnp.float32),
        pltpu.VMEM((RPT, LANES), jnp.float32),
        pltpu.VMEM_SHARED((ROWS, LANES), jnp.float32),
        pltpu.SemaphoreType.DMA,
    ],
    compiler_params=_sc_params,
)
def _sc_deg(dst_hbm, ones_hbm, zeros_hbm, out_hbm, dstv, onesv, ztile, acc, sem):
    c = lax.axis_index("c")
    s = lax.axis_index("s")
    pltpu.async_copy(dst_hbm.at[s], dstv, sem).wait()
    pltpu.async_copy(ones_hbm, onesv, sem).wait()
    pltpu.async_copy(zeros_hbm, ztile, sem).wait()
    pltpu.sync_copy(ztile, acc.at[pl.ds(s * RPT, RPT)])
    plsc.subcore_barrier()

    @pl.loop(0, CHUNKS_PER_TILE // 2)
    def _(j):
        pltpu.sync_copy(onesv, acc.at[dstv.at[2 * j + c]], add=True)

    plsc.subcore_barrier()
    pltpu.sync_copy(acc.at[pl.ds(s * RPT, RPT)],
                    out_hbm.at[c, pl.ds(s * RPT, RPT)])


@functools.partial(
    pl.kernel,
    out_type=jax.ShapeDtypeStruct((NC, ROWS, HD), jnp.float32),
    mesh=_mesh,
    scratch_types=[
        pltpu.VMEM((CHUNKS_PER_TILE, CHUNK), jnp.int32),
        pltpu.VMEM((CHUNKS_PER_TILE, CHUNK), jnp.int32),
        pltpu.VMEM((CHUNK, HD), jnp.float32),
        pltpu.VMEM((CHUNK, HD), jnp.float32),
        pltpu.VMEM((CHUNK, HD), jnp.float32),
        pltpu.VMEM((CHUNK, HD), jnp.float32),
        pltpu.VMEM((CHUNK, HD), jnp.float32),
        pltpu.VMEM_SHARED((ROWS, HD), jnp.float32),
        pltpu.SemaphoreType.DMA,
        pltpu.SemaphoreType.DMA,
        pltpu.SemaphoreType.DMA,
        pltpu.SemaphoreType.DMA,
        pltpu.SemaphoreType.DMA,
        pltpu.SemaphoreType.DMA,
        pltpu.SemaphoreType.DMA,
        pltpu.SemaphoreType.DMA,
        pltpu.SemaphoreType.DMA,
        pltpu.SemaphoreType.DMA,
        pltpu.SemaphoreType.DMA,
    ],
    compiler_params=_sc_params,
)
def _sc_agg(gh_hbm, src_hbm, dst_hbm, zrow_hbm, out_hbm,
            srcv, dstv, rows0, rows1, rows2, rows3, rows4, acc,
            sg0, sg1, sg2, sg3, sg4, ss0, ss1, ss2, ss3, ss4, semz):
    c = lax.axis_index("c")
    s = lax.axis_index("s")
    table = gh_hbm.at[c]
    bufs = (rows0, rows1, rows2, rows3, rows4)
    sgs = (sg0, sg1, sg2, sg3, sg4)
    sss = (ss0, ss1, ss2, ss3, ss4)
    pltpu.async_copy(src_hbm.at[s], srcv, semz).wait()
    pltpu.async_copy(dst_hbm.at[s], dstv, semz).wait()
    pltpu.async_copy(zrow_hbm, rows0, semz).wait()

    @pl.loop(0, RPT // CHUNK)
    def _(k):
        pltpu.sync_copy(rows0, acc.at[pl.ds(s * RPT + k * CHUNK, CHUNK)])

    plsc.subcore_barrier()

    @pl.loop(0, CHUNKS_PER_TILE + 5, step=5)
    def _(j):
        for k in range(5):
            ci = j + k
            cc = ci - 3
            kc = (k + 2) % 5

            @pl.when(ci >= 5)
            def _():
                pltpu.make_async_copy(
                    bufs[k], acc.at[dstv.at[ci - 5]], sss[k]).wait()

            @pl.when(ci < CHUNKS_PER_TILE)
            def _():
                pltpu.async_copy(table.at[srcv.at[ci]], bufs[k], sgs[k])

            @pl.when((cc >= 0) & (cc < CHUNKS_PER_TILE))
            def _():
                pltpu.make_async_copy(
                    table.at[srcv.at[cc]], bufs[kc], sgs[kc]).wait()
                pltpu.async_copy(
                    bufs[kc], acc.at[dstv.at[cc]], sss[kc], add=True)

    plsc.subcore_barrier()

    @pl.loop(0, RPT // CHUNK)
    def _(k):
        pltpu.sync_copy(acc.at[pl.ds(s * RPT + k * CHUNK, CHUNK)],
                        out_hbm.at[c, pl.ds(s * RPT + k * CHUNK, CHUNK)])


def _mm_k(x_ref, w_ref, o_ref):
    o_ref[...] = jnp.dot(x_ref[...], w_ref[...],
                         preferred_element_type=jnp.float32)


def _mm(x, w):
    return pl.pallas_call(
        _mm_k,
        out_shape=jax.ShapeDtypeStruct((x.shape[0], w.shape[1]), jnp.float32),
    )(x, w)


def _split_write(gout_ref, g):
    gout_ref[0] = g[:, :HD]
    gout_ref[1] = g[:, HD:]


def _prep_k(degp_ref, hw_ref, dinv_ref, g_ref):
    d = degp_ref[0, :N, 0:1] + degp_ref[1, :N, 0:1] + 1.0
    dinv = jax.lax.rsqrt(d)
    dinvb = jnp.broadcast_to(dinv, (N, D))
    dinv_ref[...] = dinvb
    _split_write(g_ref, dinvb * hw_ref[...])


def _prep(degp, hw):
    return pl.pallas_call(
        _prep_k,
        out_shape=[jax.ShapeDtypeStruct((N, D), jnp.float32),
                   jax.ShapeDtypeStruct((NC, N, HD), jnp.float32)],
    )(degp, hw)


def _merge(s_ref, g_ref):
    s_full = jnp.concatenate([s_ref[0, :N, :], s_ref[1, :N, :]], axis=1)
    g_full = jnp.concatenate([g_ref[0], g_ref[1]], axis=1)
    return s_full + g_full


def _layer_k(s_ref, g_ref, dinv_ref, b_ref, w_ref, gout_ref):
    agg = _merge(s_ref, g_ref)
    dinv = dinv_ref[...]
    h = jnp.maximum(dinv * agg + b_ref[...], 0.0)
    _split_write(gout_ref, dinv * jnp.dot(h, w_ref[...],
                                          preferred_element_type=jnp.float32))


def _layer(s, g, dinvb, b, w):
    return pl.pallas_call(
        _layer_k,
        out_shape=jax.ShapeDtypeStruct((NC, N, HD), jnp.float32),
    )(s, g, dinvb, b, w)


def _head_k(s_ref, g_ref, dinv_ref, b_ref, batch_ref,
            wc1_ref, bc1_ref, wc2_ref, bc2_ref, out_ref):
    agg = _merge(s_ref, g_ref)
    h = jnp.maximum(dinv_ref[...] * agg + b_ref[...], 0.0)
    b = batch_ref[...]
    gids = jax.lax.broadcasted_iota(jnp.int32, (G, N), 0)
    oh = (b[None, :] == gids).astype(jnp.float32)
    sums = jnp.dot(oh, h, preferred_element_type=jnp.float32)
    counts = jnp.sum(oh, axis=1, keepdims=True)
    pooled = sums / jnp.maximum(counts, 1.0)
    z = jnp.maximum(
        jnp.dot(pooled, wc1_ref[...], preferred_element_type=jnp.float32)
        + bc1_ref[...], 0.0)
    logits = (jnp.dot(z, wc2_ref[...], preferred_element_type=jnp.float32)
              + bc2_ref[...])
    m = jnp.max(logits, axis=1, keepdims=True)
    lse = jnp.log(jnp.sum(jnp.exp(logits - m), axis=1, keepdims=True)) + m
    out_ref[...] = logits - lse


def _head(s, g, dinvb, b, batch, wc1, bc1, wc2, bc2):
    return pl.pallas_call(
        _head_k,
        out_shape=jax.ShapeDtypeStruct((G, 2), jnp.float32),
    )(s, g, dinvb, b, batch, wc1, bc1, wc2, bc2)


def kernel(x, edge_index, batch, W1, b1, W2, b2, W3, b3, Wc1, bc1, Wc2, bc2):
    src = edge_index[0].astype(jnp.int32)
    dst = edge_index[1].astype(jnp.int32)
    src_p = jnp.concatenate(
        [src, jnp.zeros((EP - E,), jnp.int32)]
    ).reshape(NS, CHUNKS_PER_TILE, CHUNK)
    pad_dst = PAD_ROW + jnp.arange(EP - E, dtype=jnp.int32) % (ROWS - N)
    dst_p = jnp.concatenate([dst, pad_dst]).reshape(NS, CHUNKS_PER_TILE, CHUNK)
    ones16 = jnp.ones((CHUNK, LANES), jnp.float32)
    z16 = jnp.zeros((RPT, LANES), jnp.float32)
    zrow = jnp.zeros((CHUNK, HD), jnp.float32)

    degp = _sc_deg(dst_p, ones16, z16)
    hw1 = _mm(x, W1)
    dinvb, g1 = _prep(degp, hw1)
    s1 = _sc_agg(g1, src_p, dst_p, zrow)
    g2 = _layer(s1, g1, dinvb, b1, W2)
    s2 = _sc_agg(g2, src_p, dst_p, zrow)
    g3 = _layer(s2, g2, dinvb, b2, W3)
    s3 = _sc_agg(g3, src_p, dst_p, zrow)
    return _head(s3, g3, dinvb, b3, batch.astype(jnp.int32),
                 Wc1, bc1, Wc2, bc2)
